# baseline (speedup 1.0000x reference)
import functools

import jax
import jax.numpy as jnp
from jax import lax
from jax.experimental import pallas as pl
from jax.experimental.pallas import tpu as pltpu
from jax.experimental.pallas import tpu_sc as plsc

N = 10000
E = 320000
DF = 128
DE = 16
H = 128
ENC = 64

NC = 2
NS = 16
NW = NC * NS
EH = E // 2
EWH = EH // NW
CHH = 40
NCHUNKH = EWH // CHH

BE = 16000
BN = 10000

_EPS = 1e-5


def _leaky(z):
    return jnp.maximum(z, 0.01 * z)


def _stats_update(st_ref, h):
    s = jnp.sum(h, axis=0, keepdims=True)
    q = jnp.sum(h * h, axis=0, keepdims=True)
    st = jnp.concatenate([s, q, jnp.zeros((6, h.shape[1]), jnp.float32)], axis=0)

    @pl.when(pl.program_id(0) == 0)
    def _init():
        st_ref[...] = st

    @pl.when(pl.program_id(0) != 0)
    def _acc():
        st_ref[...] = st_ref[...] + st


def _edge_a_body(g_ref, ea_ref, w_ref, b_ref, h_ref, st_ref):
    z = (g_ref[...]
         + jnp.dot(ea_ref[...], w_ref[...], preferred_element_type=jnp.float32)
         + b_ref[...])
    h = _leaky(z)
    h_ref[...] = h
    _stats_update(st_ref, h)


def _edge_pass_a(g, ea, w, b):
    ne = g.shape[0]
    nb = ne // BE
    return pl.pallas_call(
        _edge_a_body,
        grid=(nb,),
        in_specs=[
            pl.BlockSpec((BE, H), lambda i: (i, 0)),
            pl.BlockSpec((BE, DE), lambda i: (i, 0)),
            pl.BlockSpec((DE, H), lambda i: (0, 0)),
            pl.BlockSpec((1, H), lambda i: (0, 0)),
        ],
        out_specs=[
            pl.BlockSpec((BE, H), lambda i: (i, 0)),
            pl.BlockSpec((8, H), lambda i: (0, 0)),
        ],
        out_shape=[
            jax.ShapeDtypeStruct((ne, H), jnp.float32),
            jax.ShapeDtypeStruct((8, H), jnp.float32),
        ],
    )(g, ea, w, b)


def _bn_mm_leaky_body(h_ref, a_ref, c_ref, w_ref, b_ref, o_ref, st_ref):
    o = h_ref[...] * a_ref[...] + c_ref[...]
    z = jnp.dot(o, w_ref[...], preferred_element_type=jnp.float32) + b_ref[...]
    h = _leaky(z)
    o_ref[...] = h
    _stats_update(st_ref, h)


def _bn_mm_leaky_stats(h, a, c, w, b, bm):
    rows, k = h.shape
    ko = w.shape[1]
    nb = rows // bm
    return pl.pallas_call(
        _bn_mm_leaky_body,
        grid=(nb,),
        in_specs=[
            pl.BlockSpec((bm, k), lambda i: (i, 0)),
            pl.BlockSpec((1, k), lambda i: (0, 0)),
            pl.BlockSpec((1, k), lambda i: (0, 0)),
            pl.BlockSpec((k, ko), lambda i: (0, 0)),
            pl.BlockSpec((1, ko), lambda i: (0, 0)),
        ],
        out_specs=[
            pl.BlockSpec((bm, ko), lambda i: (i, 0)),
            pl.BlockSpec((8, ko), lambda i: (0, 0)),
        ],
        out_shape=[
            jax.ShapeDtypeStruct((rows, ko), jnp.float32),
            jax.ShapeDtypeStruct((8, ko), jnp.float32),
        ],
    )(h, a, c, w, b)


def _bn_mm_body(h_ref, a_ref, c_ref, w_ref, b_ref, o_ref):
    o = h_ref[...] * a_ref[...] + c_ref[...]
    o_ref[...] = (jnp.dot(o, w_ref[...], preferred_element_type=jnp.float32)
                  + b_ref[...])


def _bn_mm(h, a, c, w, b, bm):
    rows, k = h.shape
    ko = w.shape[1]
    nb = rows // bm
    return pl.pallas_call(
        _bn_mm_body,
        grid=(nb,),
        in_specs=[
            pl.BlockSpec((bm, k), lambda i: (i, 0)),
            pl.BlockSpec((1, k), lambda i: (0, 0)),
            pl.BlockSpec((1, k), lambda i: (0, 0)),
            pl.BlockSpec((k, ko), lambda i: (0, 0)),
            pl.BlockSpec((1, ko), lambda i: (0, 0)),
        ],
        out_specs=pl.BlockSpec((bm, ko), lambda i: (i, 0)),
        out_shape=jax.ShapeDtypeStruct((rows, ko), jnp.float32),
    )(h, a, c, w, b)


def _mm_body(x_ref, w_ref, o_ref):
    o_ref[...] = jnp.dot(x_ref[...], w_ref[...],
                         preferred_element_type=jnp.float32)


def _mm(x, w, bm):
    rows, k = x.shape
    ko = w.shape[1]
    nb = rows // bm
    return pl.pallas_call(
        _mm_body,
        grid=(nb,),
        in_specs=[
            pl.BlockSpec((bm, k), lambda i: (i, 0)),
            pl.BlockSpec((k, ko), lambda i: (0, 0)),
        ],
        out_specs=pl.BlockSpec((bm, ko), lambda i: (i, 0)),
        out_shape=jax.ShapeDtypeStruct((rows, ko), jnp.float32),
    )(x, w)


def _node_d_body(x_ref, s_ref, s2_ref, wt_ref, wb_ref, b_ref, h_ref, st_ref):
    agg = (s_ref[0] + s_ref[1]) + (s2_ref[0] + s2_ref[1])
    z = (jnp.dot(x_ref[...], wt_ref[...], preferred_element_type=jnp.float32)
         + jnp.dot(agg, wb_ref[...], preferred_element_type=jnp.float32)
         + b_ref[...])
    h = _leaky(z)
    h_ref[...] = h
    _stats_update(st_ref, h)


def _node_pass_d(x, Sa, Sb, w2a_top, w2a_bot, b):
    nb = N // BN
    return pl.pallas_call(
        _node_d_body,
        grid=(nb,),
        in_specs=[
            pl.BlockSpec((BN, DF), lambda i: (i, 0)),
            pl.BlockSpec((2, BN, H), lambda i: (0, i, 0)),
            pl.BlockSpec((2, BN, H), lambda i: (0, i, 0)),
            pl.BlockSpec((DF, H), lambda i: (0, 0)),
            pl.BlockSpec((H, H), lambda i: (0, 0)),
            pl.BlockSpec((1, H), lambda i: (0, 0)),
        ],
        out_specs=[
            pl.BlockSpec((BN, H), lambda i: (i, 0)),
            pl.BlockSpec((8, H), lambda i: (0, 0)),
        ],
        out_shape=[
            jax.ShapeDtypeStruct((N, H), jnp.float32),
            jax.ShapeDtypeStruct((8, H), jnp.float32),
        ],
    )(x, Sa, Sb, w2a_top, w2a_bot, b)


@functools.cache
def _sc_mesh():
    return plsc.VectorSubcoreMesh(core_axis_name="c", subcore_axis_name="s",
                                  num_cores=NC, num_subcores=NS)


@functools.cache
def _sc_gather_kernel():
    return pl.kernel(
        _sc_gather_body,
        out_type=jax.ShapeDtypeStruct((EH, H), jnp.float32),
        mesh=_sc_mesh(),
        scratch_types=[
            pltpu.VMEM((NCHUNKH, CHH), jnp.int32),
            pltpu.VMEM((2, CHH, H), jnp.float32),
            pltpu.SemaphoreType.DMA,
        ],
    )


def _sc_gather(xp, row2):
    return _sc_gather_kernel()(xp, row2)


def _sc_gather_body(xp_hbm, row_hbm, out_hbm, idx_v, buf_v, gsem):
    cid = lax.axis_index("c")
    sid = lax.axis_index("s")
    wid = cid * NS + sid
    base = wid * EWH
    pltpu.sync_copy(row_hbm.at[wid], idx_v)

    pltpu.async_copy(xp_hbm.at[idx_v.at[0]], buf_v.at[0], gsem).wait()

    @pl.loop(0, NCHUNKH - 1)
    def _(j):
        slot = lax.rem(j, 2)
        nxt = lax.rem(j + 1, 2)
        cp = pltpu.async_copy(xp_hbm.at[idx_v.at[j + 1]], buf_v.at[nxt], gsem)
        pltpu.sync_copy(buf_v.at[slot], out_hbm.at[pl.ds(base + j * CHH, CHH)])
        cp.wait()

    last = NCHUNKH - 1
    pltpu.sync_copy(buf_v.at[lax.rem(last, 2)],
                    out_hbm.at[pl.ds(base + last * CHH, CHH)])


@functools.cache
def _sc_scatter_kernel():
    return pl.kernel(
        _sc_scatter_body,
        out_type=jax.ShapeDtypeStruct((NC, N, H), jnp.float32),
        mesh=_sc_mesh(),
        scratch_types=[
            pltpu.VMEM((NCHUNKH, CHH), jnp.int32),
            pltpu.VMEM((2, CHH, H), jnp.float32),
            pltpu.VMEM_SHARED((N, H), jnp.float32),
            pltpu.SemaphoreType.DMA,
        ],
    )


def _sc_scatter(vals, col2, zs):
    return _sc_scatter_kernel()(vals, col2, zs)


def _sc_scatter_body(v_hbm, col_hbm, zs_hbm, s_out, idx_v, vbuf, s_sh, sem):
    cid = lax.axis_index("c")
    sid = lax.axis_index("s")
    wid = cid * NS + sid
    base = wid * EWH

    @pl.when(sid == 0)
    def _zero_all():
        pltpu.sync_copy(zs_hbm, s_sh)

    pltpu.sync_copy(col_hbm.at[wid], idx_v)
    plsc.subcore_barrier()

    pltpu.async_copy(v_hbm.at[pl.ds(base, CHH)], vbuf.at[0], sem).wait()

    @pl.loop(0, NCHUNKH - 1)
    def _(j):
        slot = lax.rem(j, 2)
        nxt = lax.rem(j + 1, 2)
        cp = pltpu.async_copy(v_hbm.at[pl.ds(base + (j + 1) * CHH, CHH)],
                              vbuf.at[nxt], sem)
        pltpu.sync_copy(vbuf.at[slot], s_sh.at[idx_v.at[j]], add=True)
        cp.wait()

    last = NCHUNKH - 1
    pltpu.sync_copy(vbuf.at[lax.rem(last, 2)], s_sh.at[idx_v.at[last]], add=True)

    plsc.subcore_barrier()

    @pl.when(sid == 0)
    def _write_all():
        pltpu.sync_copy(s_sh, s_out.at[cid])


def _bn_stats(st, rows, gamma, beta):
    m = st[0:1] / rows
    v = st[1:2] / rows - m * m
    a = gamma.reshape(1, -1) / jnp.sqrt(v + _EPS)
    c = beta.reshape(1, -1) - m * a
    return a, c


def kernel(x, edge_index, edge_attr, u, batch, params):
    p = params
    rows = [edge_index[0][k * EH:(k + 1) * EH].reshape(NW, NCHUNKH, CHH)
            for k in range(2)]
    cols = [edge_index[1][k * EH:(k + 1) * EH].reshape(NW, NCHUNKH, CHH)
            for k in range(2)]
    eas = [edge_attr[k * EH:(k + 1) * EH] for k in range(2)]

    xp = _mm(x, p["W1a"][:DF], BN)
    gs = [_sc_gather(xp, r) for r in rows]
    wa = p["W1a"][DF:]
    ba = p["b1a"].reshape(1, H)
    aouts = [_edge_pass_a(g, ea, wa, ba) for g, ea in zip(gs, eas)]
    st1 = aouts[0][1] + aouts[1][1]

    a1, c1 = _bn_stats(st1, E, p["g1a"], p["be1a"])
    bouts = [_bn_mm_leaky_stats(h1, a1, c1, p["W1b"], p["b1b"].reshape(1, H),
                                BE) for h1, _ in aouts]
    st2 = bouts[0][1] + bouts[1][1]

    a2, c2 = _bn_stats(st2, E, p["g1b"], p["be1b"])
    out3s = [_bn_mm(h2, a2, c2, p["W1c"], p["b1c"].reshape(1, H), BE)
             for h2, _ in bouts]

    zs = jnp.zeros((N, H), jnp.float32)
    Ss = [_sc_scatter(o, cc, zs) for o, cc in zip(out3s, cols)]

    hD, st3 = _node_pass_d(x, Ss[0], Ss[1], p["W2a"][:DF], p["W2a"][DF:],
                           p["b2a"].reshape(1, H))

    a3, c3 = _bn_stats(st3, N, p["g2a"], p["be2a"])
    hE, st4 = _bn_mm_leaky_stats(hD, a3, c3, p["W2b"],
                                 p["b2b"].reshape(1, H), BN)

    a4, c4 = _bn_stats(st4, N, p["g2b"], p["be2b"])
    return _bn_mm(hE, a4, c4, p["W2c"], p["b2c"].reshape(1, ENC), BN)


# speedup vs baseline: 1.1842x; 1.1842x over previous
import functools

import jax
import jax.numpy as jnp
from jax import lax
from jax.experimental import pallas as pl
from jax.experimental.pallas import tpu as pltpu
from jax.experimental.pallas import tpu_sc as plsc

N = 10000
E = 320000
DF = 128
DE = 16
H = 128
ENC = 64

NC = 2
NS = 16
NW = NC * NS
EW = E // NW
CH = 80
NCHUNK = EW // CH

BE = 16000
BN = 10000

_EPS = 1e-5


def _leaky(z):
    return jnp.maximum(z, 0.01 * z)


def _stats_update(st_ref, h):
    s = jnp.sum(h, axis=0, keepdims=True)
    q = jnp.sum(h * h, axis=0, keepdims=True)
    st = jnp.concatenate([s, q, jnp.zeros((6, h.shape[1]), jnp.float32)], axis=0)

    @pl.when(pl.program_id(0) == 0)
    def _init():
        st_ref[...] = st

    @pl.when(pl.program_id(0) != 0)
    def _acc():
        st_ref[...] = st_ref[...] + st


def _edge_a_body(g_ref, ea_ref, w_ref, b_ref, h_ref, st_ref):
    z = (g_ref[...]
         + jnp.dot(ea_ref[...], w_ref[...], preferred_element_type=jnp.float32)
         + b_ref[...])
    h = _leaky(z)
    h_ref[...] = h
    _stats_update(st_ref, h)


def _edge_pass_a(g, ea, w, b):
    nb = E // BE
    return pl.pallas_call(
        _edge_a_body,
        grid=(nb,),
        in_specs=[
            pl.BlockSpec((BE, H), lambda i: (i, 0)),
            pl.BlockSpec((BE, DE), lambda i: (i, 0)),
            pl.BlockSpec((DE, H), lambda i: (0, 0)),
            pl.BlockSpec((1, H), lambda i: (0, 0)),
        ],
        out_specs=[
            pl.BlockSpec((BE, H), lambda i: (i, 0)),
            pl.BlockSpec((8, H), lambda i: (0, 0)),
        ],
        out_shape=[
            jax.ShapeDtypeStruct((E, H), jnp.float32),
            jax.ShapeDtypeStruct((8, H), jnp.float32),
        ],
    )(g, ea, w, b)


def _bn_mm_leaky_body(h_ref, a_ref, c_ref, w_ref, b_ref, o_ref, st_ref):
    o = h_ref[...] * a_ref[...] + c_ref[...]
    z = jnp.dot(o, w_ref[...], preferred_element_type=jnp.float32) + b_ref[...]
    h = _leaky(z)
    o_ref[...] = h
    _stats_update(st_ref, h)


def _bn_mm_leaky_stats(h, a, c, w, b, bm):
    rows, k = h.shape
    ko = w.shape[1]
    nb = rows // bm
    return pl.pallas_call(
        _bn_mm_leaky_body,
        grid=(nb,),
        in_specs=[
            pl.BlockSpec((bm, k), lambda i: (i, 0)),
            pl.BlockSpec((1, k), lambda i: (0, 0)),
            pl.BlockSpec((1, k), lambda i: (0, 0)),
            pl.BlockSpec((k, ko), lambda i: (0, 0)),
            pl.BlockSpec((1, ko), lambda i: (0, 0)),
        ],
        out_specs=[
            pl.BlockSpec((bm, ko), lambda i: (i, 0)),
            pl.BlockSpec((8, ko), lambda i: (0, 0)),
        ],
        out_shape=[
            jax.ShapeDtypeStruct((rows, ko), jnp.float32),
            jax.ShapeDtypeStruct((8, ko), jnp.float32),
        ],
    )(h, a, c, w, b)


def _bn_mm_body(h_ref, a_ref, c_ref, w_ref, b_ref, o_ref):
    o = h_ref[...] * a_ref[...] + c_ref[...]
    o_ref[...] = (jnp.dot(o, w_ref[...], preferred_element_type=jnp.float32)
                  + b_ref[...])


def _bn_mm(h, a, c, w, b, bm):
    rows, k = h.shape
    ko = w.shape[1]
    nb = rows // bm
    return pl.pallas_call(
        _bn_mm_body,
        grid=(nb,),
        in_specs=[
            pl.BlockSpec((bm, k), lambda i: (i, 0)),
            pl.BlockSpec((1, k), lambda i: (0, 0)),
            pl.BlockSpec((1, k), lambda i: (0, 0)),
            pl.BlockSpec((k, ko), lambda i: (0, 0)),
            pl.BlockSpec((1, ko), lambda i: (0, 0)),
        ],
        out_specs=pl.BlockSpec((bm, ko), lambda i: (i, 0)),
        out_shape=jax.ShapeDtypeStruct((rows, ko), jnp.float32),
    )(h, a, c, w, b)


def _mm_body(x_ref, w_ref, o_ref):
    o_ref[...] = jnp.dot(x_ref[...], w_ref[...],
                         preferred_element_type=jnp.float32)


def _mm(x, w, bm):
    rows, k = x.shape
    ko = w.shape[1]
    nb = rows // bm
    return pl.pallas_call(
        _mm_body,
        grid=(nb,),
        in_specs=[
            pl.BlockSpec((bm, k), lambda i: (i, 0)),
            pl.BlockSpec((k, ko), lambda i: (0, 0)),
        ],
        out_specs=pl.BlockSpec((bm, ko), lambda i: (i, 0)),
        out_shape=jax.ShapeDtypeStruct((rows, ko), jnp.float32),
    )(x, w)


def _node_d_body(x_ref, s_ref, wt_ref, wb_ref, b_ref, h_ref, st_ref):
    agg = s_ref[0] + s_ref[1]
    z = (jnp.dot(x_ref[...], wt_ref[...], preferred_element_type=jnp.float32)
         + jnp.dot(agg, wb_ref[...], preferred_element_type=jnp.float32)
         + b_ref[...])
    h = _leaky(z)
    h_ref[...] = h
    _stats_update(st_ref, h)


def _node_pass_d(x, S, w2a_top, w2a_bot, b):
    nb = N // BN
    return pl.pallas_call(
        _node_d_body,
        grid=(nb,),
        in_specs=[
            pl.BlockSpec((BN, DF), lambda i: (i, 0)),
            pl.BlockSpec((2, BN, H), lambda i: (0, i, 0)),
            pl.BlockSpec((DF, H), lambda i: (0, 0)),
            pl.BlockSpec((H, H), lambda i: (0, 0)),
            pl.BlockSpec((1, H), lambda i: (0, 0)),
        ],
        out_specs=[
            pl.BlockSpec((BN, H), lambda i: (i, 0)),
            pl.BlockSpec((8, H), lambda i: (0, 0)),
        ],
        out_shape=[
            jax.ShapeDtypeStruct((N, H), jnp.float32),
            jax.ShapeDtypeStruct((8, H), jnp.float32),
        ],
    )(x, S, w2a_top, w2a_bot, b)


@functools.cache
def _sc_mesh():
    return plsc.VectorSubcoreMesh(core_axis_name="c", subcore_axis_name="s",
                                  num_cores=NC, num_subcores=NS)


@functools.cache
def _sc_gather_kernel():
    return pl.kernel(
        _sc_gather_body,
        out_type=jax.ShapeDtypeStruct((E, H), jnp.float32),
        mesh=_sc_mesh(),
        scratch_types=[
            pltpu.VMEM((NCHUNK, CH), jnp.int32),
            pltpu.VMEM((2, CH, H), jnp.float32),
            pltpu.SemaphoreType.DMA,
        ],
    )


def _sc_gather(xp, row2):
    return _sc_gather_kernel()(xp, row2)


def _sc_gather_body(xp_hbm, row_hbm, out_hbm, idx_v, buf_v, gsem):
    cid = lax.axis_index("c")
    sid = lax.axis_index("s")
    wid = cid * NS + sid
    base = wid * EW
    pltpu.sync_copy(row_hbm.at[wid], idx_v)

    pltpu.async_copy(xp_hbm.at[idx_v.at[0]], buf_v.at[0], gsem).wait()

    @pl.loop(0, NCHUNK - 1)
    def _(j):
        slot = lax.rem(j, 2)
        nxt = lax.rem(j + 1, 2)
        cp = pltpu.async_copy(xp_hbm.at[idx_v.at[j + 1]], buf_v.at[nxt], gsem)
        pltpu.sync_copy(buf_v.at[slot], out_hbm.at[pl.ds(base + j * CH, CH)])
        cp.wait()

    last = NCHUNK - 1
    pltpu.sync_copy(buf_v.at[lax.rem(last, 2)],
                    out_hbm.at[pl.ds(base + last * CH, CH)])


@functools.cache
def _sc_scatter_kernel():
    return pl.kernel(
        _sc_scatter_body,
        out_type=jax.ShapeDtypeStruct((NC, N, H), jnp.float32),
        mesh=_sc_mesh(),
        scratch_types=[
            pltpu.VMEM((NCHUNK, CH), jnp.int32),
            pltpu.VMEM((2, CH, H), jnp.float32),
            pltpu.VMEM_SHARED((N, H), jnp.float32),
            pltpu.SemaphoreType.DMA,
        ],
    )


def _sc_scatter(vals, col2, zs):
    return _sc_scatter_kernel()(vals, col2, zs)


def _sc_scatter_body(v_hbm, col_hbm, zs_hbm, s_out, idx_v, vbuf, s_sh, sem):
    cid = lax.axis_index("c")
    sid = lax.axis_index("s")
    wid = cid * NS + sid
    base = wid * EW

    @pl.when(sid == 0)
    def _zero_all():
        pltpu.sync_copy(zs_hbm, s_sh)

    pltpu.sync_copy(col_hbm.at[wid], idx_v)
    plsc.subcore_barrier()

    pltpu.async_copy(v_hbm.at[pl.ds(base, CH)], vbuf.at[0], sem).wait()

    @pl.loop(0, NCHUNK - 1)
    def _(j):
        slot = lax.rem(j, 2)
        nxt = lax.rem(j + 1, 2)
        cp = pltpu.async_copy(v_hbm.at[pl.ds(base + (j + 1) * CH, CH)],
                              vbuf.at[nxt], sem)
        pltpu.sync_copy(vbuf.at[slot], s_sh.at[idx_v.at[j]], add=True)
        cp.wait()

    last = NCHUNK - 1
    pltpu.sync_copy(vbuf.at[lax.rem(last, 2)], s_sh.at[idx_v.at[last]], add=True)

    plsc.subcore_barrier()

    @pl.when(sid == 0)
    def _write_all():
        pltpu.sync_copy(s_sh, s_out.at[cid])


def _bn_stats(st, rows, gamma, beta):
    m = st[0:1] / rows
    v = st[1:2] / rows - m * m
    a = gamma.reshape(1, -1) / jnp.sqrt(v + _EPS)
    c = beta.reshape(1, -1) - m * a
    return a, c


def kernel(x, edge_index, edge_attr, u, batch, params):
    p = params
    row2 = edge_index[0].reshape(NW, NCHUNK, CH)
    col2 = edge_index[1].reshape(NW, NCHUNK, CH)

    xp = _mm(x, p["W1a"][:DF], BN)
    g = _sc_gather(xp, row2)
    h1, st1 = _edge_pass_a(g, edge_attr, p["W1a"][DF:], p["b1a"].reshape(1, H))

    a1, c1 = _bn_stats(st1, E, p["g1a"], p["be1a"])
    h2, st2 = _bn_mm_leaky_stats(h1, a1, c1, p["W1b"],
                                 p["b1b"].reshape(1, H), BE)

    a2, c2 = _bn_stats(st2, E, p["g1b"], p["be1b"])
    out3 = _bn_mm(h2, a2, c2, p["W1c"], p["b1c"].reshape(1, H), BE)

    zs = jnp.zeros((N, H), jnp.float32)
    S = _sc_scatter(out3, col2, zs)

    hD, st3 = _node_pass_d(x, S, p["W2a"][:DF], p["W2a"][DF:],
                           p["b2a"].reshape(1, H))

    a3, c3 = _bn_stats(st3, N, p["g2a"], p["be2a"])
    hE, st4 = _bn_mm_leaky_stats(hD, a3, c3, p["W2b"],
                                 p["b2b"].reshape(1, H), BN)

    a4, c4 = _bn_stats(st4, N, p["g2b"], p["be2b"])
    return _bn_mm(hE, a4, c4, p["W2c"], p["b2c"].reshape(1, ENC), BN)
